# baseline (speedup 1.0000x reference)
import jax
import jax.numpy as jnp
from jax import lax
from jax.experimental import pallas as pl
from jax.experimental.pallas import tpu as pltpu
from jax.experimental.pallas import tpu_sc as plsc

N_NODES = 10000
N_EDGES = 5000
NNZ = 320000
D = 128

NC = 2
NS = 16
NW = NC * NS
CHUNK = 128
CPW = 80
G = 16
NG = CPW // G
NNZ_PAD = NW * CPW * CHUNK
NPAD = 10240
GARBAGE = NPAD - 1

ROWS_BLK = 1000


def _mesh():
    return plsc.VectorSubcoreMesh(core_axis_name="c", subcore_axis_name="s")


def _segsum_body(tbl, sidx, didx, out,
                 sidx_v, didx_v, rows0, rows1, acc, sem0, sem1):
    cid = lax.axis_index("c")
    sid = lax.axis_index("s")
    wid = cid * NS + sid
    npr = NPAD // NS

    def zfill(i, _):
        for k in range(D // 16):
            rows0[i, pl.ds(16 * k, 16)] = jnp.zeros((16,), jnp.float32)
        return 0
    lax.fori_loop(0, CHUNK, zfill, 0)
    base = sid * npr
    for t in range(npr // CHUNK):
        pltpu.sync_copy(rows0, acc.at[pl.ds(base + t * CHUNK, CHUNK)])
    plsc.subcore_barrier()

    def group(g, _):
        pltpu.sync_copy(sidx.at[wid].at[pl.ds(g * G, G)], sidx_v)
        pltpu.sync_copy(didx.at[wid].at[pl.ds(g * G, G)], didx_v)

        pltpu.async_copy(tbl.at[sidx_v.at[0]], rows0, sem0)
        pltpu.async_copy(tbl.at[sidx_v.at[1]], rows1, sem1)

        def body(i, _):
            j0 = 2 * i
            pltpu.make_async_copy(tbl.at[sidx_v.at[j0]], rows0, sem0).wait()
            pltpu.sync_copy(rows0, acc.at[didx_v.at[j0]], add=True)

            @pl.when(i < G // 2 - 1)
            def _():
                pltpu.async_copy(tbl.at[sidx_v.at[j0 + 2]], rows0, sem0)

            pltpu.make_async_copy(tbl.at[sidx_v.at[j0 + 1]], rows1, sem1).wait()
            pltpu.sync_copy(rows1, acc.at[didx_v.at[j0 + 1]], add=True)

            @pl.when(i < G // 2 - 1)
            def _():
                pltpu.async_copy(tbl.at[sidx_v.at[j0 + 3]], rows1, sem1)
            return 0
        lax.fori_loop(0, G // 2, body, 0)
        return 0
    lax.fori_loop(0, NG, group, 0)
    plsc.subcore_barrier()

    pltpu.sync_copy(acc.at[pl.ds(base, npr)],
                    out.at[cid].at[pl.ds(base, npr)])


_segsum = pl.kernel(
    _segsum_body,
    out_type=jax.ShapeDtypeStruct((NC, NPAD, D), jnp.float32),
    mesh=_mesh(),
    scratch_types=[
        pltpu.VMEM((G, CHUNK), jnp.int32),
        pltpu.VMEM((G, CHUNK), jnp.int32),
        pltpu.VMEM((CHUNK, D), jnp.float32),
        pltpu.VMEM((CHUNK, D), jnp.float32),
        pltpu.VMEM_SHARED((NPAD, D), jnp.float32),
        pltpu.SemaphoreType.DMA,
        pltpu.SemaphoreType.DMA,
    ],
)


def _dv_scale(dvp):
    cnt = dvp[0, :, 0] + dvp[1, :, 0]
    return jnp.where(cnt > 0, lax.rsqrt(cnt), 0.0)


def _de_scale(dep):
    cnt = dep[0, :, 0] + dep[1, :, 0]
    return jnp.where(cnt > 0, 1.0 / cnt, 0.0)


def _lrelu(x):
    return jnp.where(x >= 0, x, 0.1 * x)


def _stage_a_kernel(x_ref, w_ref, b_ref, dvp_ref, xs_ref, skip_ref):
    y = lax.dot_general(x_ref[...], w_ref[...], (((1,), (0,)), ((), ())),
                        preferred_element_type=jnp.float32) + b_ref[...]
    scale = _dv_scale(dvp_ref[...])
    xs_ref[...] = y[:, :D] * scale[:, None]
    skip_ref[...] = y[:, D:]


def _stage_a(x, w_cat, b_cat, dvp):
    nb = N_NODES // ROWS_BLK
    return pl.pallas_call(
        _stage_a_kernel,
        grid=(nb,),
        in_specs=[
            pl.BlockSpec((ROWS_BLK, D), lambda i: (i, 0)),
            pl.BlockSpec((D, 2 * D), lambda i: (0, 0)),
            pl.BlockSpec((1, 2 * D), lambda i: (0, 0)),
            pl.BlockSpec((NC, ROWS_BLK, D), lambda i: (0, i, 0)),
        ],
        out_specs=[pl.BlockSpec((ROWS_BLK, D), lambda i: (i, 0)),
                   pl.BlockSpec((ROWS_BLK, D), lambda i: (i, 0))],
        out_shape=[jax.ShapeDtypeStruct((NPAD, D), jnp.float32),
                   jax.ShapeDtypeStruct((N_NODES, D), jnp.float32)],
    )(x, w_cat, b_cat, dvp)


def _stage_b_kernel(ep_ref, dep_ref, eout_ref):
    de = _de_scale(dep_ref[...])[:, None]
    e = (ep_ref[0] + ep_ref[1]) * de
    eout_ref[...] = _lrelu(e) * de


def _stage_b(ep, dep):
    nb = N_EDGES // ROWS_BLK
    return pl.pallas_call(
        _stage_b_kernel,
        grid=(nb,),
        in_specs=[
            pl.BlockSpec((NC, ROWS_BLK, D), lambda i: (0, i, 0)),
            pl.BlockSpec((NC, ROWS_BLK, D), lambda i: (0, i, 0)),
        ],
        out_specs=pl.BlockSpec((ROWS_BLK, D), lambda i: (i, 0)),
        out_shape=jax.ShapeDtypeStruct((NPAD, D), jnp.float32),
    )(ep, dep)


def _stage_c_kernel(xp_ref, skip_ref, dvp_ref, xout_ref):
    dv = _dv_scale(dvp_ref[...])[:, None]
    xn = (xp_ref[0] + xp_ref[1]) * dv + skip_ref[...]
    xout_ref[...] = _lrelu(xn)


def _stage_c(xp, skip, dvp):
    nb = N_NODES // ROWS_BLK
    return pl.pallas_call(
        _stage_c_kernel,
        grid=(nb,),
        in_specs=[
            pl.BlockSpec((NC, ROWS_BLK, D), lambda i: (0, i, 0)),
            pl.BlockSpec((ROWS_BLK, D), lambda i: (i, 0)),
            pl.BlockSpec((NC, ROWS_BLK, D), lambda i: (0, i, 0)),
        ],
        out_specs=pl.BlockSpec((ROWS_BLK, D), lambda i: (i, 0)),
        out_shape=jax.ShapeDtypeStruct((N_NODES, D), jnp.float32),
    )(xp, skip, dvp)


def _scale_in_kernel(x_ref, dvp_ref, out_ref):
    out_ref[...] = x_ref[...] * _dv_scale(dvp_ref[...])[:, None]


def _scale_in(x, dvp):
    nb = N_NODES // ROWS_BLK
    return pl.pallas_call(
        _scale_in_kernel,
        grid=(nb,),
        in_specs=[
            pl.BlockSpec((ROWS_BLK, D), lambda i: (i, 0)),
            pl.BlockSpec((NC, ROWS_BLK, D), lambda i: (0, i, 0)),
        ],
        out_specs=pl.BlockSpec((ROWS_BLK, D), lambda i: (i, 0)),
        out_shape=jax.ShapeDtypeStruct((NPAD, D), jnp.float32),
    )(x, dvp)


def _final_e_kernel(ep_ref, dep_ref, out_ref):
    out_ref[...] = (ep_ref[0] + ep_ref[1]) * _de_scale(dep_ref[...])[:, None]


def _final_e(ep, dep):
    nb = N_EDGES // ROWS_BLK
    return pl.pallas_call(
        _final_e_kernel,
        grid=(nb,),
        in_specs=[
            pl.BlockSpec((NC, ROWS_BLK, D), lambda i: (0, i, 0)),
            pl.BlockSpec((NC, ROWS_BLK, D), lambda i: (0, i, 0)),
        ],
        out_specs=pl.BlockSpec((ROWS_BLK, D), lambda i: (i, 0)),
        out_shape=jax.ShapeDtypeStruct((N_EDGES, D), jnp.float32),
    )(ep, dep)


@jax.jit
def kernel(X, node_idx, edge_idx, params):
    pad = NNZ_PAD - NNZ
    shape3 = (NW, CPW, CHUNK)
    zpad = jnp.zeros((pad,), jnp.int32)
    gpad = jnp.full((pad,), GARBAGE, jnp.int32)
    nidx_src = jnp.concatenate([node_idx, zpad]).reshape(shape3)
    eidx_src = jnp.concatenate([edge_idx, zpad]).reshape(shape3)
    nidx_dst = jnp.concatenate([node_idx, gpad]).reshape(shape3)
    eidx_dst = jnp.concatenate([edge_idx, gpad]).reshape(shape3)

    ones_tbl = jnp.ones((NPAD, D), jnp.float32)
    zidx = jnp.zeros(shape3, jnp.int32)
    dvp = _segsum(ones_tbl, zidx, nidx_dst)
    dep = _segsum(ones_tbl, zidx, eidx_dst)

    for layer in params:
        for wkey, bkey, pkey, pbkey in (("fc1_w", "fc1_b", "proj1_w", "proj1_b"),
                                        ("fc2_w", "fc2_b", "proj2_w", "proj2_b")):
            w_cat = jnp.concatenate(
                [layer[wkey].T, layer[pkey].T], axis=1)
            b_cat = jnp.concatenate(
                [layer[bkey], layer[pbkey]]).reshape(1, 2 * D)
            xs, skip = _stage_a(X, w_cat, b_cat, dvp)
            ep = _segsum(xs, nidx_src, eidx_dst)
            ein = _stage_b(ep, dep)
            xp = _segsum(ein, eidx_src, nidx_dst)
            X = _stage_c(xp, skip, dvp)

    xs_f = _scale_in(X, dvp)
    ep_f = _segsum(xs_f, nidx_src, eidx_dst)
    e_final = _final_e(ep_f, dep)
    return (e_final, X)


# speedup vs baseline: 7.3057x; 7.3057x over previous
import jax
import jax.numpy as jnp
from jax import lax
from jax.experimental import pallas as pl
from jax.experimental.pallas import tpu as pltpu
from jax.experimental.pallas import tpu_sc as plsc

N_NODES = 10000
N_EDGES = 5000
NNZ = 320000
D = 128

NC = 2
NS = 16
NW = NC * NS
CHUNK = 128
CPW = 80
G = 16
NG = CPW // G
NNZ_PAD = NW * CPW * CHUNK
NPAD = 10240
GARBAGE = NPAD - 1

ROWS_BLK = 1000


def _mesh():
    return plsc.VectorSubcoreMesh(core_axis_name="c", subcore_axis_name="s")


def _segsum_body(tbl, sidx, didx, out,
                 sidx_v, didx_v, rows0, rows1, acc, sem0, sem1):
    cid = lax.axis_index("c")
    sid = lax.axis_index("s")
    wid = cid * NS + sid
    npr = NPAD // NS

    def zfill(i, _):
        for k in range(D // 16):
            rows0[i, pl.ds(16 * k, 16)] = jnp.zeros((16,), jnp.float32)
        return 0
    lax.fori_loop(0, CHUNK, zfill, 0)
    base = sid * npr
    for t in range(npr // CHUNK):
        pltpu.sync_copy(rows0, acc.at[pl.ds(base + t * CHUNK, CHUNK)])
    plsc.subcore_barrier()

    def group(g, _):
        pltpu.sync_copy(sidx.at[wid].at[pl.ds(g * G, G)], sidx_v)
        pltpu.sync_copy(didx.at[wid].at[pl.ds(g * G, G)], didx_v)

        pltpu.async_copy(tbl.at[sidx_v.at[0]], rows0, sem0)
        pltpu.async_copy(tbl.at[sidx_v.at[1]], rows1, sem1)

        def body(i, _):
            j0 = 2 * i
            pltpu.make_async_copy(tbl.at[sidx_v.at[j0]], rows0, sem0).wait()
            pltpu.sync_copy(rows0, acc.at[didx_v.at[j0]], add=True)

            @pl.when(i < G // 2 - 1)
            def _():
                pltpu.async_copy(tbl.at[sidx_v.at[j0 + 2]], rows0, sem0)

            pltpu.make_async_copy(tbl.at[sidx_v.at[j0 + 1]], rows1, sem1).wait()
            pltpu.sync_copy(rows1, acc.at[didx_v.at[j0 + 1]], add=True)

            @pl.when(i < G // 2 - 1)
            def _():
                pltpu.async_copy(tbl.at[sidx_v.at[j0 + 3]], rows1, sem1)
            return 0
        lax.fori_loop(0, G // 2, body, 0)
        return 0
    lax.fori_loop(0, NG, group, 0)
    plsc.subcore_barrier()

    pltpu.sync_copy(acc.at[pl.ds(base, npr)],
                    out.at[cid].at[pl.ds(base, npr)])


_segsum = pl.kernel(
    _segsum_body,
    out_type=jax.ShapeDtypeStruct((NC, NPAD, D), jnp.float32),
    mesh=_mesh(),
    scratch_types=[
        pltpu.VMEM((G, CHUNK), jnp.int32),
        pltpu.VMEM((G, CHUNK), jnp.int32),
        pltpu.VMEM((CHUNK, D), jnp.float32),
        pltpu.VMEM((CHUNK, D), jnp.float32),
        pltpu.VMEM_SHARED((NPAD, D), jnp.float32),
        pltpu.SemaphoreType.DMA,
        pltpu.SemaphoreType.DMA,
    ],
)


def _ones_scatter_body(didx, out, didx_v, rows0, acc):
    cid = lax.axis_index("c")
    sid = lax.axis_index("s")
    wid = cid * NS + sid
    npr = NPAD // NS

    def zfill(i, _):
        for k in range(D // 16):
            rows0[i, pl.ds(16 * k, 16)] = jnp.zeros((16,), jnp.float32)
        return 0
    lax.fori_loop(0, CHUNK, zfill, 0)
    base = sid * npr
    for t in range(npr // CHUNK):
        pltpu.sync_copy(rows0, acc.at[pl.ds(base + t * CHUNK, CHUNK)])

    def ofill(i, _):
        for k in range(D // 16):
            rows0[i, pl.ds(16 * k, 16)] = jnp.ones((16,), jnp.float32)
        return 0
    lax.fori_loop(0, CHUNK, ofill, 0)
    plsc.subcore_barrier()

    def group(g, _):
        pltpu.sync_copy(didx.at[wid].at[pl.ds(g * G, G)], didx_v)

        def body(j, _):
            pltpu.sync_copy(rows0, acc.at[didx_v.at[j]], add=True)
            return 0
        lax.fori_loop(0, G, body, 0)
        return 0
    lax.fori_loop(0, NG, group, 0)
    plsc.subcore_barrier()

    pltpu.sync_copy(acc.at[pl.ds(base, npr)],
                    out.at[cid].at[pl.ds(base, npr)])


_ones_scatter = pl.kernel(
    _ones_scatter_body,
    out_type=jax.ShapeDtypeStruct((NC, NPAD, D), jnp.float32),
    mesh=_mesh(),
    scratch_types=[
        pltpu.VMEM((G, CHUNK), jnp.int32),
        pltpu.VMEM((CHUNK, D), jnp.float32),
        pltpu.VMEM_SHARED((NPAD, D), jnp.float32),
    ],
)


def _dv_scale(dvp):
    cnt = dvp[0, :, 0] + dvp[1, :, 0]
    return jnp.where(cnt > 0, lax.rsqrt(cnt), 0.0)


def _de_scale(dep):
    cnt = dep[0, :, 0] + dep[1, :, 0]
    return jnp.where(cnt > 0, 1.0 / cnt, 0.0)


def _lrelu(x):
    return jnp.where(x >= 0, x, 0.1 * x)


def _stage_a_kernel(x_ref, w_ref, b_ref, dvp_ref, xs_ref, skip_ref):
    y = lax.dot_general(x_ref[...], w_ref[...], (((1,), (0,)), ((), ())),
                        preferred_element_type=jnp.float32) + b_ref[...]
    scale = _dv_scale(dvp_ref[...])
    xs_ref[...] = y[:, :D] * scale[:, None]
    skip_ref[...] = y[:, D:]


def _stage_a(x, w_cat, b_cat, dvp):
    nb = N_NODES // ROWS_BLK
    return pl.pallas_call(
        _stage_a_kernel,
        grid=(nb,),
        in_specs=[
            pl.BlockSpec((ROWS_BLK, D), lambda i: (i, 0)),
            pl.BlockSpec((D, 2 * D), lambda i: (0, 0)),
            pl.BlockSpec((1, 2 * D), lambda i: (0, 0)),
            pl.BlockSpec((NC, ROWS_BLK, D), lambda i: (0, i, 0)),
        ],
        out_specs=[pl.BlockSpec((ROWS_BLK, D), lambda i: (i, 0)),
                   pl.BlockSpec((ROWS_BLK, D), lambda i: (i, 0))],
        out_shape=[jax.ShapeDtypeStruct((NPAD, D), jnp.float32),
                   jax.ShapeDtypeStruct((N_NODES, D), jnp.float32)],
    )(x, w_cat, b_cat, dvp)


def _stage_b_kernel(ep_ref, dep_ref, eout_ref):
    de = _de_scale(dep_ref[...])[:, None]
    e = (ep_ref[0] + ep_ref[1]) * de
    eout_ref[...] = _lrelu(e) * de


def _stage_b(ep, dep):
    nb = N_EDGES // ROWS_BLK
    return pl.pallas_call(
        _stage_b_kernel,
        grid=(nb,),
        in_specs=[
            pl.BlockSpec((NC, ROWS_BLK, D), lambda i: (0, i, 0)),
            pl.BlockSpec((NC, ROWS_BLK, D), lambda i: (0, i, 0)),
        ],
        out_specs=pl.BlockSpec((ROWS_BLK, D), lambda i: (i, 0)),
        out_shape=jax.ShapeDtypeStruct((NPAD, D), jnp.float32),
    )(ep, dep)


def _stage_c_kernel(xp_ref, skip_ref, dvp_ref, xout_ref):
    dv = _dv_scale(dvp_ref[...])[:, None]
    xn = (xp_ref[0] + xp_ref[1]) * dv + skip_ref[...]
    xout_ref[...] = _lrelu(xn)


def _stage_c(xp, skip, dvp):
    nb = N_NODES // ROWS_BLK
    return pl.pallas_call(
        _stage_c_kernel,
        grid=(nb,),
        in_specs=[
            pl.BlockSpec((NC, ROWS_BLK, D), lambda i: (0, i, 0)),
            pl.BlockSpec((ROWS_BLK, D), lambda i: (i, 0)),
            pl.BlockSpec((NC, ROWS_BLK, D), lambda i: (0, i, 0)),
        ],
        out_specs=pl.BlockSpec((ROWS_BLK, D), lambda i: (i, 0)),
        out_shape=jax.ShapeDtypeStruct((N_NODES, D), jnp.float32),
    )(xp, skip, dvp)


def _scale_in_kernel(x_ref, dvp_ref, out_ref):
    out_ref[...] = x_ref[...] * _dv_scale(dvp_ref[...])[:, None]


def _scale_in(x, dvp):
    nb = N_NODES // ROWS_BLK
    return pl.pallas_call(
        _scale_in_kernel,
        grid=(nb,),
        in_specs=[
            pl.BlockSpec((ROWS_BLK, D), lambda i: (i, 0)),
            pl.BlockSpec((NC, ROWS_BLK, D), lambda i: (0, i, 0)),
        ],
        out_specs=pl.BlockSpec((ROWS_BLK, D), lambda i: (i, 0)),
        out_shape=jax.ShapeDtypeStruct((NPAD, D), jnp.float32),
    )(x, dvp)


def _final_e_kernel(ep_ref, dep_ref, out_ref):
    out_ref[...] = (ep_ref[0] + ep_ref[1]) * _de_scale(dep_ref[...])[:, None]


def _final_e(ep, dep):
    nb = N_EDGES // ROWS_BLK
    return pl.pallas_call(
        _final_e_kernel,
        grid=(nb,),
        in_specs=[
            pl.BlockSpec((NC, ROWS_BLK, D), lambda i: (0, i, 0)),
            pl.BlockSpec((NC, ROWS_BLK, D), lambda i: (0, i, 0)),
        ],
        out_specs=pl.BlockSpec((ROWS_BLK, D), lambda i: (i, 0)),
        out_shape=jax.ShapeDtypeStruct((N_EDGES, D), jnp.float32),
    )(ep, dep)


@jax.jit
def kernel(X, node_idx, edge_idx, params):
    pad = NNZ_PAD - NNZ
    shape3 = (NW, CPW, CHUNK)
    zpad = jnp.zeros((pad,), jnp.int32)
    gpad = jnp.full((pad,), GARBAGE, jnp.int32)
    nidx_src = jnp.concatenate([node_idx, zpad]).reshape(shape3)
    eidx_src = jnp.concatenate([edge_idx, zpad]).reshape(shape3)
    nidx_dst = jnp.concatenate([node_idx, gpad]).reshape(shape3)
    eidx_dst = jnp.concatenate([edge_idx, gpad]).reshape(shape3)

    dvp = _ones_scatter(nidx_dst)
    dep = _ones_scatter(eidx_dst)

    for layer in params:
        for wkey, bkey, pkey, pbkey in (("fc1_w", "fc1_b", "proj1_w", "proj1_b"),
                                        ("fc2_w", "fc2_b", "proj2_w", "proj2_b")):
            w_cat = jnp.concatenate(
                [layer[wkey].T, layer[pkey].T], axis=1)
            b_cat = jnp.concatenate(
                [layer[bkey], layer[pbkey]]).reshape(1, 2 * D)
            xs, skip = _stage_a(X, w_cat, b_cat, dvp)
            ep = _segsum(xs, nidx_src, eidx_dst)
            ein = _stage_b(ep, dep)
            xp = _segsum(ein, eidx_src, nidx_dst)
            X = _stage_c(xp, skip, dvp)

    xs_f = _scale_in(X, dvp)
    ep_f = _segsum(xs_f, nidx_src, eidx_dst)
    e_final = _final_e(ep_f, dep)
    return (e_final, X)
